# baseline (speedup 1.0000x reference)
import functools

import jax
import jax.numpy as jnp
import numpy as np
from jax import lax
from jax.experimental import pallas as pl
from jax.experimental.pallas import tpu as pltpu
from jax.experimental.pallas import tpu_sc as plsc

B = 16384
NF = 26
VOCAB = 1000
ED = 128
IN_DIM = NF * ED
H1 = 1024
H2 = 512
EPS = 1e-5
_ISQRT = float(1.0 / np.sqrt(1.0 + EPS))

NC = 2
NS = 16
NW = NC * NS
N_SLICES = 4


def _sc_gather(featT_flat, table_flat, nbase, nb):
    mesh = plsc.VectorSubcoreMesh(core_axis_name="c", subcore_axis_name="s")
    nb_per_w = nb // NW
    SCH = min(256, nb_per_w)
    cpf = nb_per_w // SCH
    n_ch = NF * cpf

    @functools.partial(
        pl.kernel,
        mesh=mesh,
        out_type=jax.ShapeDtypeStruct((NF, nb // 2, ED), jnp.uint32),
        scratch_types=[
            pltpu.VMEM((SCH,), jnp.int32),
            pltpu.VMEM((SCH,), jnp.int32),
            pltpu.VMEM((SCH, ED), jnp.uint32),
            pltpu.VMEM((SCH, ED), jnp.uint32),
            pltpu.VMEM((SCH // 2, ED), jnp.uint32),
            pltpu.VMEM((SCH // 2, ED), jnp.uint32),
            pltpu.SemaphoreType.DMA,
            pltpu.SemaphoreType.DMA,
            pltpu.SemaphoreType.DMA,
            pltpu.SemaphoreType.DMA,
        ],
    )
    def k(feat_hbm, tab_hbm, out_hbm, idx0, idx1, rows0, rows1,
          pk0, pk1, g0, g1, w0, w1):
        wid = lax.axis_index("s") * NC + lax.axis_index("c")
        sbase = wid * nb_per_w
        sbase2 = wid * (nb_per_w // 2)

        def srcoff(c):
            return (c // cpf) * B + nbase + sbase + (c % cpf) * SCH

        def dst(c):
            off = sbase2 + (c % cpf) * (SCH // 2)
            return out_hbm.at[c // cpf, pl.ds(off, SCH // 2)]

        def wait_wb(pk, sem):
            pltpu.make_async_copy(
                pk, out_hbm.at[0, pl.ds(sbase2, SCH // 2)], sem).wait()

        _HALF = jnp.uint32(0x8000)
        _HIMASK = jnp.uint32(0xFFFF0000)
        _SH = jnp.uint32(16)

        def pack_rows(rows, pk):
            for p in range(SCH // 2):
                for j in range(ED // 16):
                    va = rows[2 * p, pl.ds(j * 16, 16)] + _HALF
                    vb = rows[2 * p + 1, pl.ds(j * 16, 16)] + _HALF
                    w = (va >> _SH) | (vb & _HIMASK)
                    pk[p, pl.ds(j * 16, 16)] = w

        def pair_body(p, carry):
            c0 = 2 * p
            c1 = 2 * p + 1
            pltpu.sync_copy(feat_hbm.at[pl.ds(srcoff(c0), SCH)], idx0)
            pltpu.sync_copy(feat_hbm.at[pl.ds(srcoff(c1), SCH)], idx1)
            for buf, c in ((idx0, c0), (idx1, c1)):
                voff = (c // cpf) * VOCAB
                for j in range(SCH // 16):
                    v = buf[pl.ds(j * 16, 16)]
                    v = jnp.minimum(jnp.maximum(v, 0), VOCAB - 1) + voff
                    buf[pl.ds(j * 16, 16)] = v

            cp0 = pltpu.async_copy(tab_hbm.at[idx0], rows0, g0)
            cp1 = pltpu.async_copy(tab_hbm.at[idx1], rows1, g1)
            cp0.wait()

            @pl.when(p > 0)
            def _():
                wait_wb(pk0, w0)

            pack_rows(rows0, pk0)
            pltpu.async_copy(pk0, dst(c0), w0)
            cp1.wait()

            @pl.when(p > 0)
            def _():
                wait_wb(pk1, w1)

            pack_rows(rows1, pk1)
            pltpu.async_copy(pk1, dst(c1), w1)
            return carry

        lax.fori_loop(0, n_ch // 2, pair_body, 0)
        wait_wb(pk0, w0)
        wait_wb(pk1, w1)

    return k(featT_flat, table_flat)


def _mlp(xt, gamma2, beta2, w1, b1r, w2, b2r, w3, b3r, nb):
    BLK = 256
    grid = (nb // BLK,)

    def body(xt_ref, g_ref, be_ref, w1_ref, b1_ref, w2_ref, b2_ref,
             w3_ref, b3_ref, o_ref):
        xb = jnp.concatenate(
            [pltpu.bitcast(xt_ref[f], jnp.bfloat16) for f in range(NF)],
            axis=-1)
        gs = (g_ref[...] * _ISQRT).astype(jnp.bfloat16)
        xb = xb * gs + be_ref[...].astype(jnp.bfloat16)
        h = jnp.dot(xb, w1_ref[...].astype(jnp.bfloat16),
                    preferred_element_type=jnp.float32)
        h = jnp.maximum(h + b1_ref[...], 0.0)
        h = jnp.dot(h.astype(jnp.bfloat16), w2_ref[...].astype(jnp.bfloat16),
                    preferred_element_type=jnp.float32)
        h = jnp.maximum(h + b2_ref[...], 0.0)
        o = jnp.dot(h, w3_ref[...], preferred_element_type=jnp.float32)
        o_ref[...] = jax.nn.sigmoid(o + b3_ref[...])

    out = pl.pallas_call(
        body,
        grid=grid,
        in_specs=[
            pl.BlockSpec((NF, BLK // 2, ED), lambda i: (0, i, 0)),
            pl.BlockSpec((1, IN_DIM), lambda i: (0, 0)),
            pl.BlockSpec((1, IN_DIM), lambda i: (0, 0)),
            pl.BlockSpec((IN_DIM, H1), lambda i: (0, 0)),
            pl.BlockSpec((1, H1), lambda i: (0, 0)),
            pl.BlockSpec((H1, H2), lambda i: (0, 0)),
            pl.BlockSpec((1, H2), lambda i: (0, 0)),
            pl.BlockSpec((H2, 1), lambda i: (0, 0)),
            pl.BlockSpec((1, 1), lambda i: (0, 0)),
        ],
        out_specs=pl.BlockSpec((BLK, 1), lambda i: (i, 0)),
        out_shape=jax.ShapeDtypeStruct((nb, 1), jnp.float32),
    )(xt, gamma2, beta2, w1, b1r, w2, b2r, w3, b3r)
    return out[:, 0]


def kernel(features, tables, gamma, beta, W1, b1, W2, b2, W3, b3):
    featT_flat = features.astype(jnp.int32).T.reshape(NF * B)
    table_flat = lax.bitcast_convert_type(
        tables.reshape(NF * VOCAB, ED), jnp.uint32)
    g2 = gamma.reshape(1, IN_DIM)
    be2 = beta.reshape(1, IN_DIM)
    b1r = b1.reshape(1, H1)
    b2r = b2.reshape(1, H2)
    b3r = b3.reshape(1, 1)
    nb = B // N_SLICES
    outs = []
    for s in range(N_SLICES):
        xt = _sc_gather(featT_flat, table_flat, s * nb, nb)
        outs.append(_mlp(xt, g2, be2, W1, b1r, W2, b2r, W3, b3r, nb))
    return jnp.concatenate(outs)


# speedup vs baseline: 1.4450x; 1.4450x over previous
import functools

import jax
import jax.numpy as jnp
import numpy as np
from jax import lax
from jax.experimental import pallas as pl
from jax.experimental.pallas import tpu as pltpu
from jax.experimental.pallas import tpu_sc as plsc

B = 16384
NF = 26
VOCAB = 1000
ED = 128
IN_DIM = NF * ED
H1 = 1024
H2 = 512
EPS = 1e-5
_ISQRT = float(1.0 / np.sqrt(1.0 + EPS))

NC = 2
NS = 16
NW = NC * NS
N_SLICES = 4


def _sc_gather(featJ, table_flat, nb):
    mesh = plsc.VectorSubcoreMesh(core_axis_name="c", subcore_axis_name="s")
    npw = nb // NW
    jobs = NF * npw
    N_CH = 8
    CHR = jobs // N_CH
    assert CHR % 16 == 0 and npw % 16 == 0

    def segments(c):
        segs, r = [], c * CHR
        while r < (c + 1) * CHR:
            f = r // npw
            end = min((f + 1) * npw, (c + 1) * CHR)
            segs.append((r - c * CHR, f, r - f * npw, end - r))
            r = end
        return segs

    @functools.partial(
        pl.kernel,
        mesh=mesh,
        out_type=jax.ShapeDtypeStruct((NF, nb, ED), jnp.float32),
        scratch_types=[
            pltpu.VMEM((CHR,), jnp.int32),
            pltpu.VMEM((CHR,), jnp.int32),
            pltpu.VMEM((CHR, ED), jnp.float32),
            pltpu.VMEM((CHR, ED), jnp.float32),
            pltpu.SemaphoreType.DMA,
            pltpu.SemaphoreType.DMA,
            pltpu.SemaphoreType.DMA,
            pltpu.SemaphoreType.DMA,
        ],
    )
    def k(feat_hbm, tab_hbm, out_hbm, idx0, idx1, rows0, rows1, g0, g1, w0, w1):
        wid = lax.axis_index("s") * NC + lax.axis_index("c")
        jbase = wid * jobs
        sbase = wid * npw
        idx = (idx0, idx1)
        rows = (rows0, rows1)
        gsem = (g0, g1)
        wsem = (w0, w1)

        def wait_wb(s):
            pltpu.make_async_copy(
                rows[s], out_hbm.at[0, pl.ds(0, CHR)], wsem[s]).wait()

        def start_chunk(c):
            s = c % 2
            pltpu.sync_copy(feat_hbm.at[pl.ds(jbase + c * CHR, CHR)], idx[s])
            for j in range(CHR // 16):
                voff = ((c * CHR + j * 16) // npw) * VOCAB
                v = idx[s][pl.ds(j * 16, 16)]
                v = jnp.minimum(jnp.maximum(v, 0), VOCAB - 1) + voff
                idx[s][pl.ds(j * 16, 16)] = v
            return pltpu.async_copy(tab_hbm.at[idx[s]], rows[s], gsem[s])

        def writeback_chunk(c):
            s = c % 2
            for lo, f, slo, ln in segments(c):
                pltpu.async_copy(
                    rows[s].at[pl.ds(lo, ln)],
                    out_hbm.at[f, pl.ds(sbase + slo, ln)],
                    wsem[s])

        cps = {}
        for c in range(N_CH):
            s = c % 2
            if c >= 2:
                wait_wb(s)
            cps[c] = start_chunk(c)
            if c >= 1:
                cps[c - 1].wait()
                writeback_chunk(c - 1)
        cps[N_CH - 1].wait()
        writeback_chunk(N_CH - 1)
        wait_wb(0)
        wait_wb(1)

    return k(featJ, table_flat)


def _mlp(xt, gamma2, beta2, w1, b1r, w2, b2r, w3, b3r, nb):
    BLK = 256
    grid = (nb // BLK,)

    def body(xt_ref, g_ref, be_ref, w1_ref, b1_ref, w2_ref, b2_ref,
             w3_ref, b3_ref, o_ref):
        xb = jnp.concatenate([xt_ref[f] for f in range(NF)], axis=-1)
        xb = xb * (g_ref[...] * _ISQRT) + be_ref[...]
        h = jnp.dot(xb.astype(jnp.bfloat16), w1_ref[...].astype(jnp.bfloat16),
                    preferred_element_type=jnp.float32)
        h = jnp.maximum(h + b1_ref[...], 0.0)
        h = jnp.dot(h.astype(jnp.bfloat16), w2_ref[...].astype(jnp.bfloat16),
                    preferred_element_type=jnp.float32)
        h = jnp.maximum(h + b2_ref[...], 0.0)
        o = jnp.dot(h, w3_ref[...], preferred_element_type=jnp.float32)
        o_ref[...] = jax.nn.sigmoid(o + b3_ref[...])

    out = pl.pallas_call(
        body,
        grid=grid,
        in_specs=[
            pl.BlockSpec((NF, BLK, ED), lambda i: (0, i, 0)),
            pl.BlockSpec((1, IN_DIM), lambda i: (0, 0)),
            pl.BlockSpec((1, IN_DIM), lambda i: (0, 0)),
            pl.BlockSpec((IN_DIM, H1), lambda i: (0, 0)),
            pl.BlockSpec((1, H1), lambda i: (0, 0)),
            pl.BlockSpec((H1, H2), lambda i: (0, 0)),
            pl.BlockSpec((1, H2), lambda i: (0, 0)),
            pl.BlockSpec((H2, 1), lambda i: (0, 0)),
            pl.BlockSpec((1, 1), lambda i: (0, 0)),
        ],
        out_specs=pl.BlockSpec((BLK, 1), lambda i: (i, 0)),
        out_shape=jax.ShapeDtypeStruct((nb, 1), jnp.float32),
    )(xt, gamma2, beta2, w1, b1r, w2, b2r, w3, b3r)
    return out[:, 0]


def kernel(features, tables, gamma, beta, W1, b1, W2, b2, W3, b3):
    nb = B // N_SLICES
    npw = nb // NW
    featJ = (features.astype(jnp.int32).T
             .reshape(NF, N_SLICES, NW, npw)
             .transpose(1, 2, 0, 3)
             .reshape(N_SLICES, NW * NF * npw))
    table_flat = tables.reshape(NF * VOCAB, ED)
    g2 = gamma.reshape(1, IN_DIM)
    be2 = beta.reshape(1, IN_DIM)
    b1r = b1.reshape(1, H1)
    b2r = b2.reshape(1, H2)
    b3r = b3.reshape(1, 1)
    outs = []
    for s in range(N_SLICES):
        xt = _sc_gather(featJ[s], table_flat, nb)
        outs.append(_mlp(xt, g2, be2, W1, b1r, W2, b2r, W3, b3r, nb))
    return jnp.concatenate(outs)


# speedup vs baseline: 1.4997x; 1.0379x over previous
import functools

import jax
import jax.numpy as jnp
import numpy as np
from jax import lax
from jax.experimental import pallas as pl
from jax.experimental.pallas import tpu as pltpu
from jax.experimental.pallas import tpu_sc as plsc

B = 16384
NF = 26
VOCAB = 1000
ED = 128
IN_DIM = NF * ED
H1 = 1024
H2 = 512
EPS = 1e-5
_ISQRT = float(1.0 / np.sqrt(1.0 + EPS))

NC = 2
NS = 16
NW = NC * NS
N_SLICES = 4


def _sc_gather(featT_flat, table_flat, nbase, nb):
    mesh = plsc.VectorSubcoreMesh(core_axis_name="c", subcore_axis_name="s")
    nb_per_w = nb // NW
    SCH = min(256, nb_per_w)
    cpf = nb_per_w // SCH
    n_ch = NF * cpf

    @functools.partial(
        pl.kernel,
        mesh=mesh,
        out_type=jax.ShapeDtypeStruct((NF, nb, ED), jnp.float32),
        scratch_types=[
            pltpu.VMEM((SCH,), jnp.int32),
            pltpu.VMEM((SCH,), jnp.int32),
            pltpu.VMEM((SCH, ED), jnp.float32),
            pltpu.VMEM((SCH, ED), jnp.float32),
            pltpu.SemaphoreType.DMA,
            pltpu.SemaphoreType.DMA,
            pltpu.SemaphoreType.DMA,
            pltpu.SemaphoreType.DMA,
        ],
    )
    def k(feat_hbm, tab_hbm, out_hbm, idx0, idx1, rows0, rows1, g0, g1, w0, w1):
        wid = lax.axis_index("s") * NC + lax.axis_index("c")
        sbase = wid * nb_per_w

        def srcoff(c):
            return (c // cpf) * B + nbase + sbase + (c % cpf) * SCH

        def dst(c):
            return out_hbm.at[c // cpf, pl.ds(sbase + (c % cpf) * SCH, SCH)]

        def wait_wb(rows, sem):
            pltpu.make_async_copy(
                rows, out_hbm.at[0, pl.ds(sbase, SCH)], sem).wait()

        def pair_body(p, carry):
            c0 = 2 * p
            c1 = 2 * p + 1
            pltpu.sync_copy(feat_hbm.at[pl.ds(srcoff(c0), SCH)], idx0)
            pltpu.sync_copy(feat_hbm.at[pl.ds(srcoff(c1), SCH)], idx1)
            for buf, c in ((idx0, c0), (idx1, c1)):
                voff = (c // cpf) * VOCAB
                for j in range(SCH // 16):
                    v = buf[pl.ds(j * 16, 16)]
                    v = jnp.minimum(jnp.maximum(v, 0), VOCAB - 1) + voff
                    buf[pl.ds(j * 16, 16)] = v

            @pl.when(p > 0)
            def _():
                wait_wb(rows0, w0)

            cp0 = pltpu.async_copy(tab_hbm.at[idx0], rows0, g0)

            @pl.when(p > 0)
            def _():
                wait_wb(rows1, w1)

            cp1 = pltpu.async_copy(tab_hbm.at[idx1], rows1, g1)
            cp0.wait()
            pltpu.async_copy(rows0, dst(c0), w0)
            cp1.wait()
            pltpu.async_copy(rows1, dst(c1), w1)
            return carry

        lax.fori_loop(0, n_ch // 2, pair_body, 0)
        wait_wb(rows0, w0)
        wait_wb(rows1, w1)

    return k(featT_flat, table_flat)


def _mlp(xt, gamma2, beta2, w1, b1r, w2, b2r, w3, b3r, nb):
    BLK = 256
    grid = (nb // BLK,)

    def body(xt_ref, g_ref, be_ref, w1_ref, b1_ref, w2_ref, b2_ref,
             w3_ref, b3_ref, o_ref):
        xb = jnp.concatenate([xt_ref[f] for f in range(NF)], axis=-1)
        xb = xb * (g_ref[...] * _ISQRT) + be_ref[...]
        h = jnp.dot(xb.astype(jnp.bfloat16), w1_ref[...].astype(jnp.bfloat16),
                    preferred_element_type=jnp.float32)
        h = jnp.maximum(h + b1_ref[...], 0.0)
        h = jnp.dot(h.astype(jnp.bfloat16), w2_ref[...].astype(jnp.bfloat16),
                    preferred_element_type=jnp.float32)
        h = jnp.maximum(h + b2_ref[...], 0.0)
        o = jnp.dot(h, w3_ref[...], preferred_element_type=jnp.float32)
        o_ref[...] = jax.nn.sigmoid(o + b3_ref[...])

    out = pl.pallas_call(
        body,
        grid=grid,
        in_specs=[
            pl.BlockSpec((NF, BLK, ED), lambda i: (0, i, 0)),
            pl.BlockSpec((1, IN_DIM), lambda i: (0, 0)),
            pl.BlockSpec((1, IN_DIM), lambda i: (0, 0)),
            pl.BlockSpec((IN_DIM, H1), lambda i: (0, 0)),
            pl.BlockSpec((1, H1), lambda i: (0, 0)),
            pl.BlockSpec((H1, H2), lambda i: (0, 0)),
            pl.BlockSpec((1, H2), lambda i: (0, 0)),
            pl.BlockSpec((H2, 1), lambda i: (0, 0)),
            pl.BlockSpec((1, 1), lambda i: (0, 0)),
        ],
        out_specs=pl.BlockSpec((BLK, 1), lambda i: (i, 0)),
        out_shape=jax.ShapeDtypeStruct((nb, 1), jnp.float32),
    )(xt, gamma2, beta2, w1, b1r, w2, b2r, w3, b3r)
    return out[:, 0]


def kernel(features, tables, gamma, beta, W1, b1, W2, b2, W3, b3):
    featT_flat = features.astype(jnp.int32).T.reshape(NF * B)
    table_flat = tables.reshape(NF * VOCAB, ED)
    g2 = gamma.reshape(1, IN_DIM)
    be2 = beta.reshape(1, IN_DIM)
    b1r = b1.reshape(1, H1)
    b2r = b2.reshape(1, H2)
    b3r = b3.reshape(1, 1)
    nb = B // N_SLICES
    outs = []
    for s in range(N_SLICES):
        xt = _sc_gather(featT_flat, table_flat, s * nb, nb)
        outs.append(_mlp(xt, g2, be2, W1, b1r, W2, b2r, W3, b3r, nb))
    return jnp.concatenate(outs)
